# baseline (speedup 1.0000x reference)
import functools

import jax
import jax.numpy as jnp
from jax.experimental import pallas as pl
from jax.experimental.pallas import tpu as pltpu
from jax.experimental.pallas import tpu_sc as plsc

N = 50000
K = 16
NG = 32
DE = 12
DF = 64
DP = 64

B = 512
N_PAD = 50176
PAD = N_PAD - N
GRID = N_PAD // B
EB = B * K
E_PAD = N_PAD * K

PB = 2000
PGRID = N // PB

NW = 32
PER_W = E_PAD // NW
CHUNK = 3136


def _pack_body(idx_ref, pc_ref, table_ref, pack_ref):
    idx = idx_ref[0]
    cats = jax.lax.broadcasted_iota(jnp.int32, (39, 1), 0)
    oh = (cats == idx).astype(jnp.float32)
    attr = jax.lax.dot_general(
        oh, table_ref[...], (((0,), (0,)), ((), ())),
        preferred_element_type=jnp.float32)
    pad = jnp.zeros((PB, 1), jnp.float32)
    pack_ref[...] = jnp.concatenate([pc_ref[...], pad, attr], axis=-1)


def _pack(pc, table, attr_idx):
    return pl.pallas_call(
        _pack_body,
        grid=(PGRID,),
        in_specs=[
            pl.BlockSpec((1, 1, PB), lambda i: (i, 0, 0)),
            pl.BlockSpec((PB, 3), lambda i: (i, 0)),
            pl.BlockSpec((39, DE), lambda i: (0, 0)),
        ],
        out_specs=pl.BlockSpec((PB, 16), lambda i: (i, 0)),
        out_shape=jax.ShapeDtypeStruct((N, 16), jnp.float32),
    )(attr_idx.reshape(PGRID, 1, PB), pc, table)


def _gather(pack, nbr_flat):
    mesh = plsc.VectorSubcoreMesh(core_axis_name="c", subcore_axis_name="s")

    @functools.partial(
        pl.kernel,
        mesh=mesh,
        out_type=jax.ShapeDtypeStruct((E_PAD, 16), jnp.float32),
        scratch_types=[
            pltpu.VMEM((CHUNK,), jnp.int32),
            pltpu.VMEM((CHUNK,), jnp.int32),
            pltpu.VMEM((CHUNK, 16), jnp.float32),
            pltpu.VMEM((CHUNK, 16), jnp.float32),
            pltpu.SemaphoreType.DMA,
            pltpu.SemaphoreType.DMA,
            pltpu.SemaphoreType.DMA,
        ],
        compiler_params=pltpu.CompilerParams(use_tc_tiling_on_sc=False),
    )
    def k(pack_hbm, idx_hbm, out_hbm, i0, i1, r0, r1, sg, so0, so1):
        wid = jax.lax.axis_index("s") * 2 + jax.lax.axis_index("c")
        base = wid * PER_W
        idxs, rows, sos = (i0, i1), (r0, r1), (so0, so1)
        outcp = [None, None]
        for c in range(PER_W // CHUNK):
            b = c & 1
            if outcp[b] is not None:
                outcp[b].wait()
            pltpu.sync_copy(idx_hbm.at[pl.ds(base + c * CHUNK, CHUNK)],
                            idxs[b])
            pltpu.async_copy(pack_hbm.at[idxs[b]], rows[b], sg).wait()
            outcp[b] = pltpu.async_copy(
                rows[b], out_hbm.at[pl.ds(base + c * CHUNK, CHUNK)], sos[b])
        outcp[0].wait()
        outcp[1].wait()

    return k(pack, nbr_flat)


def _main_body(gath_ref, pcT_ref, wcT_ref, wattT_ref, wfT_ref, coefT_ref,
               centT_ref, m_ref, out_ref, psum_ref, psq_ref):
    i = pl.program_id(0)
    blk = gath_ref[...]
    blkT = blk.T
    x16T = jnp.concatenate([blkT[16 * j:16 * (j + 1), :] for j in range(8)],
                           axis=1)
    pcnT = x16T[0:3, :]
    pcrT = jnp.concatenate([pcT_ref[...]] * K, axis=1)
    relT = pcnT - pcrT
    d2 = jnp.dot(m_ref[...], relT * relT,
                 preferred_element_type=jnp.float32)
    d = jnp.sqrt(d2 + 1e-6)
    gT = jnp.exp(coefT_ref[...] * (d - centT_ref[...]) ** 2)
    xT = jnp.concatenate([gT, x16T], axis=0)
    yT = jnp.maximum(
        jnp.dot(wcT_ref[...], xT, preferred_element_type=jnp.float32), 0.0)
    lg = jnp.dot(wattT_ref[...], yT,
                 preferred_element_type=jnp.float32)
    u = jnp.exp(lg)
    wT = yT * u
    t = wT[:, 0:B]
    den = u[:, 0:B]
    for k in range(1, K):
        t = t + wT[:, k * B:(k + 1) * B]
        den = den + u[:, k * B:(k + 1) * B]
    pooledT = jnp.dot(wfT_ref[...], t / den,
                      preferred_element_type=jnp.float32)
    pooled = pooledT.T
    rows = jax.lax.broadcasted_iota(jnp.int32, (B, 1), 0)
    valid = jnp.where(i == GRID - 1, B - PAD, B)
    pm = pooled * (rows < valid).astype(jnp.float32)
    out_ref[...] = pooled
    psum_ref[...] = jnp.sum(pm, axis=0, keepdims=True).reshape(1, 1, DP)
    psq_ref[...] = jnp.sum(pm * pm, axis=0,
                           keepdims=True).reshape(1, 1, DP)


def _main(gathered, pcT, wcT, wattT, wfT, coefT, centT, msk):
    return pl.pallas_call(
        _main_body,
        grid=(GRID,),
        in_specs=[
            pl.BlockSpec((EB // 8, 128), lambda i: (i, 0)),
            pl.BlockSpec((3, B), lambda i: (0, i)),
            pl.BlockSpec((DF, 48), lambda i: (0, 0)),
            pl.BlockSpec((1, DF), lambda i: (0, 0)),
            pl.BlockSpec((DP, DF), lambda i: (0, 0)),
            pl.BlockSpec((NG, 1), lambda i: (0, 0)),
            pl.BlockSpec((NG, 1), lambda i: (0, 0)),
            pl.BlockSpec((1, 3), lambda i: (0, 0)),
        ],
        out_specs=[
            pl.BlockSpec((B, DP), lambda i: (i, 0)),
            pl.BlockSpec((1, 1, DP), lambda i: (i, 0, 0)),
            pl.BlockSpec((1, 1, DP), lambda i: (i, 0, 0)),
        ],
        out_shape=[
            jax.ShapeDtypeStruct((N_PAD, DP), jnp.float32),
            jax.ShapeDtypeStruct((GRID, 1, DP), jnp.float32),
            jax.ShapeDtypeStruct((GRID, 1, DP), jnp.float32),
        ],
    )(gathered, pcT, wcT, wattT, wfT, coefT, centT, msk)


def _bn_body(x_ref, ps_ref, pq_ref, gamma_ref, beta_ref, out_ref):
    s = jnp.sum(ps_ref[...], axis=0)
    q = jnp.sum(pq_ref[...], axis=0)
    mean = s / N
    var = q / N - mean * mean
    inv = jax.lax.rsqrt(var + 1e-5)
    out_ref[...] = jnp.maximum(
        (x_ref[...] - mean) * inv * gamma_ref[...] + beta_ref[...], 0.0)


def _bn(pooled_pad, psum, psq, gamma, beta):
    return pl.pallas_call(
        _bn_body,
        grid=(PGRID,),
        in_specs=[
            pl.BlockSpec((PB, DP), lambda i: (i, 0)),
            pl.BlockSpec((GRID, 1, DP), lambda i: (0, 0, 0)),
            pl.BlockSpec((GRID, 1, DP), lambda i: (0, 0, 0)),
            pl.BlockSpec((1, DP), lambda i: (0, 0)),
            pl.BlockSpec((1, DP), lambda i: (0, 0)),
        ],
        out_specs=pl.BlockSpec((PB, DP), lambda i: (i, 0)),
        out_shape=jax.ShapeDtypeStruct((N, DP), jnp.float32),
    )(pooled_pad, psum, psq, gamma.reshape(1, DP), beta.reshape(1, DP))


@jax.jit
def kernel(pc, table, centers, sigmas, W1, W2, W_att, W_feat, gamma, beta,
           attr_idx, nbr):
    pack = _pack(pc, table, attr_idx.astype(jnp.int32))
    nbr_pad = jnp.concatenate(
        [nbr.reshape(-1).astype(jnp.int32), jnp.zeros(PAD * K, jnp.int32)])
    idxp = (nbr_pad.reshape(GRID, B, 8, 2)
            .transpose(0, 3, 1, 2).reshape(-1))
    gathered = _gather(pack, idxp)
    gathered = gathered.reshape(E_PAD // 8, 128)
    wc = jnp.concatenate([W1, jnp.zeros((4, DF), jnp.float32), W2], axis=0)
    pcT = jnp.concatenate([pc, jnp.zeros((PAD, 3), jnp.float32)], axis=0).T
    coefT = (-0.5 / (sigmas * sigmas)).reshape(NG, 1)
    centT = centers.reshape(NG, 1)
    pooled_pad, psum, psq = _main(
        gathered, pcT, wc.T, W_att.T, W_feat.T, coefT, centT,
        jnp.ones((1, 3), jnp.float32))
    return _bn(pooled_pad, psum, psq, gamma, beta)


# speedup vs baseline: 1.8987x; 1.8987x over previous
import functools

import jax
import jax.numpy as jnp
import numpy as _np
from jax.experimental import pallas as pl
from jax.experimental.pallas import tpu as pltpu
from jax.experimental.pallas import tpu_sc as plsc

N = 50000
K = 16
NG = 32
DE = 12
DF = 64
DP = 64

B = 512
N_PAD = 50176
PAD = N_PAD - N
GRID = N_PAD // B
EB = B * K
E_PAD = N_PAD * K

PB = 2000
PGRID = N // PB

NW = 32
PER_W = E_PAD // NW
CHUNK = 3136

_p = _np.arange(EB)
_q = (_p % 8) * (EB // 8) + _p // 8
_PERM = ((_q % B) * K + _q // B).astype(_np.int32)


def _pack_body(idx_ref, pc_ref, table_ref, pack_ref):
    idx = idx_ref[0]
    cats = jax.lax.broadcasted_iota(jnp.int32, (39, 1), 0)
    oh = (cats == idx).astype(jnp.float32)
    attr = jax.lax.dot_general(
        oh, table_ref[...], (((0,), (0,)), ((), ())),
        preferred_element_type=jnp.float32)
    pad = jnp.zeros((PB, 1), jnp.float32)
    pack_ref[...] = jnp.concatenate([pc_ref[...], pad, attr], axis=-1)


def _pack(pc, table, attr_idx):
    return pl.pallas_call(
        _pack_body,
        grid=(PGRID,),
        in_specs=[
            pl.BlockSpec((1, 1, PB), lambda i: (i, 0, 0)),
            pl.BlockSpec((PB, 3), lambda i: (i, 0)),
            pl.BlockSpec((39, DE), lambda i: (0, 0)),
        ],
        out_specs=pl.BlockSpec((PB, 16), lambda i: (i, 0)),
        out_shape=jax.ShapeDtypeStruct((N, 16), jnp.float32),
    )(attr_idx.reshape(PGRID, 1, PB), pc, table)


def _gather(pack, nbr_flat):
    mesh = plsc.VectorSubcoreMesh(core_axis_name="c", subcore_axis_name="s")

    @functools.partial(
        pl.kernel,
        mesh=mesh,
        out_type=jax.ShapeDtypeStruct((E_PAD, 16), jnp.float32),
        scratch_types=[
            pltpu.VMEM((CHUNK,), jnp.int32),
            pltpu.VMEM((CHUNK,), jnp.int32),
            pltpu.VMEM((CHUNK, 16), jnp.float32),
            pltpu.VMEM((CHUNK, 16), jnp.float32),
            pltpu.SemaphoreType.DMA,
            pltpu.SemaphoreType.DMA,
            pltpu.SemaphoreType.DMA,
        ],
        compiler_params=pltpu.CompilerParams(use_tc_tiling_on_sc=False),
    )
    def k(pack_hbm, idx_hbm, out_hbm, i0, i1, r0, r1, sg, so0, so1):
        wid = jax.lax.axis_index("s") * 2 + jax.lax.axis_index("c")
        base = wid * PER_W
        idxs, rows, sos = (i0, i1), (r0, r1), (so0, so1)
        outcp = [None, None]
        for c in range(PER_W // CHUNK):
            b = c & 1
            if outcp[b] is not None:
                outcp[b].wait()
            pltpu.sync_copy(idx_hbm.at[pl.ds(base + c * CHUNK, CHUNK)],
                            idxs[b])
            pltpu.async_copy(pack_hbm.at[idxs[b]], rows[b], sg).wait()
            outcp[b] = pltpu.async_copy(
                rows[b], out_hbm.at[pl.ds(base + c * CHUNK, CHUNK)], sos[b])
        outcp[0].wait()
        outcp[1].wait()

    return k(pack, nbr_flat)


def _main_body(gath_ref, pcT_ref, wcT_ref, wattT_ref, wfT_ref, coefT_ref,
               centT_ref, m_ref, out_ref, psum_ref, psq_ref):
    i = pl.program_id(0)
    blk = gath_ref[...]
    blkT = blk.T
    x16T = jnp.concatenate([blkT[16 * j:16 * (j + 1), :] for j in range(8)],
                           axis=1)
    pcnT = x16T[0:3, :]
    pcrT = jnp.concatenate([pcT_ref[...]] * K, axis=1)
    relT = pcnT - pcrT
    d2 = jnp.dot(m_ref[...], relT * relT,
                 preferred_element_type=jnp.float32)
    d = jnp.sqrt(d2 + 1e-6)
    gT = jnp.exp(coefT_ref[...] * (d - centT_ref[...]) ** 2)
    xT = jnp.concatenate([gT, x16T], axis=0)
    yT = jnp.maximum(
        jnp.dot(wcT_ref[...], xT, preferred_element_type=jnp.float32), 0.0)
    lg = jnp.dot(wattT_ref[...], yT,
                 preferred_element_type=jnp.float32)
    u = jnp.exp(lg)
    wT = yT * u
    t = wT[:, 0:B]
    den = u[:, 0:B]
    for k in range(1, K):
        t = t + wT[:, k * B:(k + 1) * B]
        den = den + u[:, k * B:(k + 1) * B]
    pooledT = jnp.dot(wfT_ref[...], t / den,
                      preferred_element_type=jnp.float32)
    pooled = pooledT.T
    rows = jax.lax.broadcasted_iota(jnp.int32, (B, 1), 0)
    valid = jnp.where(i == GRID - 1, B - PAD, B)
    pm = pooled * (rows < valid).astype(jnp.float32)
    out_ref[...] = pooled
    psum_ref[...] = jnp.sum(pm, axis=0, keepdims=True).reshape(1, 1, DP)
    psq_ref[...] = jnp.sum(pm * pm, axis=0,
                           keepdims=True).reshape(1, 1, DP)


def _main(gathered, pcT, wcT, wattT, wfT, coefT, centT, msk):
    return pl.pallas_call(
        _main_body,
        grid=(GRID,),
        in_specs=[
            pl.BlockSpec((EB // 8, 128), lambda i: (i, 0)),
            pl.BlockSpec((3, B), lambda i: (0, i)),
            pl.BlockSpec((DF, 48), lambda i: (0, 0)),
            pl.BlockSpec((1, DF), lambda i: (0, 0)),
            pl.BlockSpec((DP, DF), lambda i: (0, 0)),
            pl.BlockSpec((NG, 1), lambda i: (0, 0)),
            pl.BlockSpec((NG, 1), lambda i: (0, 0)),
            pl.BlockSpec((1, 3), lambda i: (0, 0)),
        ],
        out_specs=[
            pl.BlockSpec((B, DP), lambda i: (i, 0)),
            pl.BlockSpec((1, 1, DP), lambda i: (i, 0, 0)),
            pl.BlockSpec((1, 1, DP), lambda i: (i, 0, 0)),
        ],
        out_shape=[
            jax.ShapeDtypeStruct((N_PAD, DP), jnp.float32),
            jax.ShapeDtypeStruct((GRID, 1, DP), jnp.float32),
            jax.ShapeDtypeStruct((GRID, 1, DP), jnp.float32),
        ],
    )(gathered, pcT, wcT, wattT, wfT, coefT, centT, msk)


def _bn_body(x_ref, ps_ref, pq_ref, gamma_ref, beta_ref, out_ref):
    s = jnp.sum(ps_ref[...], axis=0)
    q = jnp.sum(pq_ref[...], axis=0)
    mean = s / N
    var = q / N - mean * mean
    inv = jax.lax.rsqrt(var + 1e-5)
    out_ref[...] = jnp.maximum(
        (x_ref[...] - mean) * inv * gamma_ref[...] + beta_ref[...], 0.0)


def _bn(pooled_pad, psum, psq, gamma, beta):
    return pl.pallas_call(
        _bn_body,
        grid=(PGRID,),
        in_specs=[
            pl.BlockSpec((PB, DP), lambda i: (i, 0)),
            pl.BlockSpec((GRID, 1, DP), lambda i: (0, 0, 0)),
            pl.BlockSpec((GRID, 1, DP), lambda i: (0, 0, 0)),
            pl.BlockSpec((1, DP), lambda i: (0, 0)),
            pl.BlockSpec((1, DP), lambda i: (0, 0)),
        ],
        out_specs=pl.BlockSpec((PB, DP), lambda i: (i, 0)),
        out_shape=jax.ShapeDtypeStruct((N, DP), jnp.float32),
    )(pooled_pad, psum, psq, gamma.reshape(1, DP), beta.reshape(1, DP))


@jax.jit
def kernel(pc, table, centers, sigmas, W1, W2, W_att, W_feat, gamma, beta,
           attr_idx, nbr):
    pack = _pack(pc, table, attr_idx.astype(jnp.int32))
    nbr_pad = jnp.concatenate(
        [nbr.reshape(-1).astype(jnp.int32), jnp.zeros(PAD * K, jnp.int32)])
    idxp = jnp.stack([nbr_pad[0::2].reshape(GRID, EB // 2),
                      nbr_pad[1::2].reshape(GRID, EB // 2)],
                     axis=1).reshape(-1)
    gathered = _gather(pack, idxp)
    gathered = gathered.reshape(E_PAD // 8, 128)
    wc = jnp.concatenate([W1, jnp.zeros((4, DF), jnp.float32), W2], axis=0)
    pcT = jnp.concatenate([pc, jnp.zeros((PAD, 3), jnp.float32)], axis=0).T
    coefT = (-0.5 / (sigmas * sigmas)).reshape(NG, 1)
    centT = centers.reshape(NG, 1)
    pooled_pad, psum, psq = _main(
        gathered, pcT, wc.T, W_att.T, W_feat.T, coefT, centT,
        jnp.ones((1, 3), jnp.float32))
    return _bn(pooled_pad, psum, psq, gamma, beta)


# speedup vs baseline: 2.4819x; 1.3072x over previous
import functools

import jax
import jax.numpy as jnp
import numpy as _np
from jax.experimental import pallas as pl
from jax.experimental.pallas import tpu as pltpu
from jax.experimental.pallas import tpu_sc as plsc

N = 50000
K = 16
NG = 32
DE = 12
DF = 64
DP = 64

B = 512
N_PAD = 50176
PAD = N_PAD - N
GRID = N_PAD // B
EB = B * K
E_PAD = N_PAD * K

PB = 2000
PGRID = N // PB

NW = 32
PER_W = E_PAD // NW
CHUNK = 3136

_p = _np.arange(EB)
_q = (_p % 8) * (EB // 8) + _p // 8
_PERM = ((_q % B) * K + _q // B).astype(_np.int32)


def _pack_body(idx_ref, pc_ref, table_ref, pack_ref):
    idx = idx_ref[0]
    cats = jax.lax.broadcasted_iota(jnp.int32, (39, 1), 0)
    oh = (cats == idx).astype(jnp.float32)
    attr = jax.lax.dot_general(
        oh, table_ref[...], (((0,), (0,)), ((), ())),
        preferred_element_type=jnp.float32)
    pad = jnp.zeros((PB, 1), jnp.float32)
    pack_ref[...] = jnp.concatenate([pc_ref[...], pad, attr], axis=-1)


def _pack(pc, table, attr_idx):
    return pl.pallas_call(
        _pack_body,
        grid=(PGRID,),
        in_specs=[
            pl.BlockSpec((1, 1, PB), lambda i: (i, 0, 0)),
            pl.BlockSpec((PB, 3), lambda i: (i, 0)),
            pl.BlockSpec((39, DE), lambda i: (0, 0)),
        ],
        out_specs=pl.BlockSpec((PB, 16), lambda i: (i, 0)),
        out_shape=jax.ShapeDtypeStruct((N, 16), jnp.float32),
    )(attr_idx.reshape(PGRID, 1, PB), pc, table)


def _gather(pack, nbr_flat):
    mesh = plsc.VectorSubcoreMesh(core_axis_name="c", subcore_axis_name="s")

    @functools.partial(
        pl.kernel,
        mesh=mesh,
        out_type=jax.ShapeDtypeStruct((E_PAD, 16), jnp.float32),
        scratch_types=[
            pltpu.VMEM((CHUNK,), jnp.int32),
            pltpu.VMEM((CHUNK,), jnp.int32),
            pltpu.VMEM((CHUNK, 16), jnp.float32),
            pltpu.VMEM((CHUNK, 16), jnp.float32),
            pltpu.SemaphoreType.DMA,
            pltpu.SemaphoreType.DMA,
            pltpu.SemaphoreType.DMA,
        ],
        compiler_params=pltpu.CompilerParams(use_tc_tiling_on_sc=False),
    )
    def k(pack_hbm, idx_hbm, out_hbm, i0, i1, r0, r1, sg, so0, so1):
        wid = jax.lax.axis_index("s") * 2 + jax.lax.axis_index("c")
        base = wid * PER_W
        idxs, rows, sos = (i0, i1), (r0, r1), (so0, so1)
        outcp = [None, None]
        for c in range(PER_W // CHUNK):
            b = c & 1
            if outcp[b] is not None:
                outcp[b].wait()
            pltpu.sync_copy(idx_hbm.at[pl.ds(base + c * CHUNK, CHUNK)],
                            idxs[b])
            pltpu.async_copy(pack_hbm.at[idxs[b]], rows[b], sg).wait()
            outcp[b] = pltpu.async_copy(
                rows[b], out_hbm.at[pl.ds(base + c * CHUNK, CHUNK)], sos[b])
        outcp[0].wait()
        outcp[1].wait()

    return k(pack, nbr_flat)


def _main_body(gath_ref, pcT_ref, wcT_ref, wattT_ref, wfT_ref, coefT_ref,
               centT_ref, m_ref, out_ref, psum_ref, psq_ref):
    i = pl.program_id(0)
    blk = gath_ref[...]
    blkT = blk.T
    x16T = jnp.concatenate([blkT[16 * j:16 * (j + 1), :] for j in range(8)],
                           axis=1)
    pcnT = x16T[0:3, :]
    pcrT = jnp.concatenate([pcT_ref[...]] * K, axis=1)
    relT = pcnT - pcrT
    d2 = jnp.dot(m_ref[...], relT * relT,
                 preferred_element_type=jnp.float32)
    d = jnp.sqrt(d2 + 1e-6)
    gT = jnp.exp(coefT_ref[...] * (d - centT_ref[...]) ** 2)
    xT = jnp.concatenate([gT, x16T], axis=0)
    yT = jnp.maximum(
        jnp.dot(wcT_ref[...], xT, preferred_element_type=jnp.float32), 0.0)
    lg = jnp.dot(wattT_ref[...], yT,
                 preferred_element_type=jnp.float32)
    u = jnp.exp(lg)
    wT = yT * u
    t = wT[:, 0:B]
    den = u[:, 0:B]
    for k in range(1, K):
        t = t + wT[:, k * B:(k + 1) * B]
        den = den + u[:, k * B:(k + 1) * B]
    pooledT = jnp.dot(wfT_ref[...], t / den,
                      preferred_element_type=jnp.float32)
    pooled = pooledT.T
    rows = jax.lax.broadcasted_iota(jnp.int32, (B, 1), 0)
    valid = jnp.where(i == GRID - 1, B - PAD, B)
    pm = pooled * (rows < valid).astype(jnp.float32)
    out_ref[...] = pooled
    psum_ref[...] = jnp.sum(pm, axis=0, keepdims=True).reshape(1, 1, DP)
    psq_ref[...] = jnp.sum(pm * pm, axis=0,
                           keepdims=True).reshape(1, 1, DP)


def _main(gathered, pcT, wcT, wattT, wfT, coefT, centT, msk):
    return pl.pallas_call(
        _main_body,
        grid=(GRID,),
        in_specs=[
            pl.BlockSpec((EB // 8, 128), lambda i: (i, 0)),
            pl.BlockSpec((3, B), lambda i: (0, i)),
            pl.BlockSpec((DF, 48), lambda i: (0, 0)),
            pl.BlockSpec((1, DF), lambda i: (0, 0)),
            pl.BlockSpec((DP, DF), lambda i: (0, 0)),
            pl.BlockSpec((NG, 1), lambda i: (0, 0)),
            pl.BlockSpec((NG, 1), lambda i: (0, 0)),
            pl.BlockSpec((1, 3), lambda i: (0, 0)),
        ],
        out_specs=[
            pl.BlockSpec((B, DP), lambda i: (i, 0)),
            pl.BlockSpec((1, 1, DP), lambda i: (i, 0, 0)),
            pl.BlockSpec((1, 1, DP), lambda i: (i, 0, 0)),
        ],
        out_shape=[
            jax.ShapeDtypeStruct((N_PAD, DP), jnp.float32),
            jax.ShapeDtypeStruct((GRID, 1, DP), jnp.float32),
            jax.ShapeDtypeStruct((GRID, 1, DP), jnp.float32),
        ],
    )(gathered, pcT, wcT, wattT, wfT, coefT, centT, msk)


def _bn_body(x_ref, ps_ref, pq_ref, gamma_ref, beta_ref, out_ref):
    s = jnp.sum(ps_ref[...], axis=0)
    q = jnp.sum(pq_ref[...], axis=0)
    mean = s / N
    var = q / N - mean * mean
    inv = jax.lax.rsqrt(var + 1e-5)
    out_ref[...] = jnp.maximum(
        (x_ref[...] - mean) * inv * gamma_ref[...] + beta_ref[...], 0.0)


def _bn(pooled_pad, psum, psq, gamma, beta):
    return pl.pallas_call(
        _bn_body,
        grid=(PGRID,),
        in_specs=[
            pl.BlockSpec((PB, DP), lambda i: (i, 0)),
            pl.BlockSpec((GRID, 1, DP), lambda i: (0, 0, 0)),
            pl.BlockSpec((GRID, 1, DP), lambda i: (0, 0, 0)),
            pl.BlockSpec((1, DP), lambda i: (0, 0)),
            pl.BlockSpec((1, DP), lambda i: (0, 0)),
        ],
        out_specs=pl.BlockSpec((PB, DP), lambda i: (i, 0)),
        out_shape=jax.ShapeDtypeStruct((N, DP), jnp.float32),
    )(pooled_pad, psum, psq, gamma.reshape(1, DP), beta.reshape(1, DP))


@jax.jit
def kernel(pc, table, centers, sigmas, W1, W2, W_att, W_feat, gamma, beta,
           attr_idx, nbr):
    pack = _pack(pc, table, attr_idx.astype(jnp.int32))
    nbr_pad = jnp.concatenate(
        [nbr.reshape(-1).astype(jnp.int32), jnp.zeros(PAD * K, jnp.int32)])
    idxp = nbr_pad.reshape(GRID, EB)[:, _PERM].reshape(-1)
    gathered = _gather(pack, idxp)
    gathered = gathered.reshape(E_PAD // 8, 128)
    wc = jnp.concatenate([W1, jnp.zeros((4, DF), jnp.float32), W2], axis=0)
    pcT = jnp.concatenate([pc, jnp.zeros((PAD, 3), jnp.float32)], axis=0).T
    coefT = (-0.5 / (sigmas * sigmas)).reshape(NG, 1)
    centT = centers.reshape(NG, 1)
    pooled_pad, psum, psq = _main(
        gathered, pcT, wc.T, W_att.T, W_feat.T, coefT, centT,
        jnp.ones((1, 3), jnp.float32))
    return _bn(pooled_pad, psum, psq, gamma, beta)
